# Initial kernel scaffold; baseline (speedup 1.0000x reference)
import jax
import jax.numpy as jnp
from jax.experimental import pallas as pl


def kernel(x, c, r, edge_index_xx, edge_index_cx, edge_index_rx, W_x, b_x, W_c, b_c, W_r, b_r, W_xx, b_xx, W_cx, b_cx, W_rx, b_rx, W_pool, b_pool):
    raise NotImplementedError("write your pallas kernel here")


# speedup vs baseline: 9.9721x; 9.9721x over previous
import functools

import jax
import jax.numpy as jnp
from jax import lax
from jax.experimental import pallas as pl
from jax.experimental.pallas import tpu as pltpu
from jax.experimental.pallas import tpu_sc as plsc

N = 10000
D = 128
H = 64
OUT = 128
NT = 3 * N
E_TOT = 640000
NCORE = 2
NSUB = 16
NW = NCORE * NSUB
CHUNK = 128
EPW = -(-E_TOT // (NW * CHUNK)) * CHUNK
E_PAD = EPW * NW
NCHUNKS = EPW // CHUNK
AGG_R = 10112
RPT = AGG_R // NSUB
BM = 1000


def _transform_body(x_ref, w_ref, b_ref, g_ref, s_ref):
    res = jnp.dot(x_ref[...], w_ref[0], preferred_element_type=jnp.float32)
    res = jnp.maximum(res + b_ref[0], 0.0)
    g_ref[...] = res[:, :H]
    s_ref[...] = res[:, H:]


def _transform(X3, Wcat, Bcat):
    per_rel = N // BM
    return pl.pallas_call(
        _transform_body,
        grid=(NT // BM,),
        in_specs=[
            pl.BlockSpec((BM, D), lambda i: (i, 0)),
            pl.BlockSpec((1, D, 2 * H), lambda i: (i // per_rel, 0, 0)),
            pl.BlockSpec((1, 1, 2 * H), lambda i: (i // per_rel, 0, 0)),
        ],
        out_specs=[
            pl.BlockSpec((BM, H), lambda i: (i, 0)),
            pl.BlockSpec((BM, H), lambda i: (i, 0)),
        ],
        out_shape=[
            jax.ShapeDtypeStruct((NT, H), jnp.float32),
            jax.ShapeDtypeStruct((NT, H), jnp.float32),
        ],
    )(X3, Wcat, Bcat)


_mesh = plsc.VectorSubcoreMesh(core_axis_name="c", subcore_axis_name="s")


@functools.partial(
    pl.kernel,
    out_type=jax.ShapeDtypeStruct((NCORE, AGG_R, H), jnp.float32),
    mesh=_mesh,
    scratch_types=[
        pltpu.VMEM((CHUNK,), jnp.int32),
        pltpu.VMEM((CHUNK,), jnp.int32),
        pltpu.VMEM((CHUNK, H), jnp.float32),
        pltpu.VMEM_SHARED((AGG_R, H), jnp.float32),
    ],
    compiler_params=pltpu.CompilerParams(use_tc_tiling_on_sc=False),
)
def _sc_agg(g_hbm, src_hbm, dst_hbm, zero_hbm, out_hbm, src_v, dst_v, rows_v,
            agg_sh):
    cid = lax.axis_index("c")
    sid = lax.axis_index("s")
    wid = sid * NCORE + cid
    pltpu.sync_copy(zero_hbm.at[pl.ds(sid * RPT, RPT)],
                    agg_sh.at[pl.ds(sid * RPT, RPT)])
    plsc.subcore_barrier()

    def body(k, carry):
        base = wid * EPW + k * CHUNK
        pltpu.sync_copy(src_hbm.at[pl.ds(base, CHUNK)], src_v)
        pltpu.sync_copy(dst_hbm.at[pl.ds(base, CHUNK)], dst_v)
        pltpu.sync_copy(g_hbm.at[src_v], rows_v)
        pltpu.sync_copy(rows_v, agg_sh.at[dst_v], add=True)
        return carry

    lax.fori_loop(0, NCHUNKS, body, 0)
    plsc.subcore_barrier()
    pltpu.sync_copy(agg_sh.at[pl.ds(sid * RPT, RPT)],
                    out_hbm.at[cid, pl.ds(sid * RPT, RPT)])


def _pool_body(a_ref, s_ref, wp_ref, bp_ref, o_ref):
    acc = a_ref[0] + a_ref[1] + s_ref[...]
    o_ref[...] = (jnp.dot(acc, wp_ref[...], preferred_element_type=jnp.float32)
                  + bp_ref[...])


def _pool(agg, S3, W_pool, b_pool2):
    return pl.pallas_call(
        _pool_body,
        grid=(N // BM,),
        in_specs=[
            pl.BlockSpec((NCORE, BM, H), lambda i: (0, i, 0)),
            pl.BlockSpec((BM, H), lambda i: (i, 0)),
            pl.BlockSpec((H, OUT), lambda i: (0, 0)),
            pl.BlockSpec((1, OUT), lambda i: (0, 0)),
        ],
        out_specs=pl.BlockSpec((BM, OUT), lambda i: (i, 0)),
        out_shape=jax.ShapeDtypeStruct((N, OUT), jnp.float32),
    )(agg, S3, W_pool, b_pool2)


def kernel(x, c, r, edge_index_xx, edge_index_cx, edge_index_rx,
           W_x, b_x, W_c, b_c, W_r, b_r,
           W_xx, b_xx, W_cx, b_cx, W_rx, b_rx,
           W_pool, b_pool):
    X3 = jnp.concatenate([x, c, r], axis=0)
    Wcat = jnp.stack([
        jnp.concatenate([W_xx, W_x], axis=1),
        jnp.concatenate([W_cx, W_c], axis=1),
        jnp.concatenate([W_rx, W_r], axis=1),
    ])
    Bcat = jnp.stack([
        jnp.concatenate([b_xx, b_x]),
        jnp.concatenate([b_cx, b_c]),
        jnp.concatenate([b_rx, b_r]),
    ])[:, None, :]
    G, S3 = _transform(X3, Wcat, Bcat)

    pad = E_PAD - E_TOT
    i32 = jnp.int32
    src = jnp.concatenate([
        edge_index_xx[0].astype(i32),
        edge_index_cx[0].astype(i32) + N,
        edge_index_rx[0].astype(i32) + 2 * N,
        jnp.zeros((pad,), i32),
    ])
    dst = jnp.concatenate([
        edge_index_xx[1].astype(i32),
        edge_index_cx[1].astype(i32),
        edge_index_rx[1].astype(i32),
        jnp.full((pad,), N, i32),
    ])
    zeros = jnp.zeros((AGG_R, H), jnp.float32)

    agg = _sc_agg(G, src, dst, zeros)

    x_out = _pool(agg, S3, W_pool, b_pool[None, :])
    c_out = lax.slice_in_dim(S3, N, 2 * N, axis=0)
    r_out = lax.slice_in_dim(S3, 2 * N, 3 * N, axis=0)
    return (x_out, c_out, r_out)
